# baseline (speedup 1.0000x reference)
import math

import jax
import jax.numpy as jnp
from jax.experimental import pallas as pl
from jax.experimental.pallas import tpu as pltpu

B = 32
N = 256
D_X = 50
D_T = 14
D_OUT = 64
HALF = D_T // 2
_FREQ_SCALE = -(math.log(10000.0) / (HALF - 1))

BB = 16


def _fwd_kernel(t_ref, x_ref, w1_ref, b1_ref, o_ref):
    step = pl.program_id(0)
    w1 = w1_ref[...]
    wx = w1[:, :D_X]
    wt = w1[:, D_X:]
    b1_col = jnp.transpose(b1_ref[...], (1, 0))
    i = jax.lax.broadcasted_iota(jnp.int32, (HALF, 1), 0).astype(jnp.float32)
    freqs = jnp.exp(i * _FREQ_SCALE)
    y = jax.lax.dot_general(
        wx, x_ref[...].reshape(D_X, BB * N), (((1,), (0,)), ((), ())),
        preferred_element_type=jnp.float32,
    ).reshape(D_OUT, BB, N)
    for j in range(BB):
        tb = t_ref[step * BB + j].astype(jnp.float32)
        args = tb * freqs
        te = jnp.concatenate([jnp.sin(args), jnp.cos(args)], axis=0)
        cc = (
            jax.lax.dot_general(
                wt, te, (((1,), (0,)), ((), ())),
                preferred_element_type=jnp.float32,
            )
            + b1_col
        )
        o_ref[j, :, :] = y[:, j, :] + cc


def kernel(x, t, obj_cond, edge_cond_in, relation_cond_in, W1, b1):
    xT = jnp.transpose(x, (2, 0, 1))
    outp = pl.pallas_call(
        _fwd_kernel,
        grid=(B // BB,),
        in_specs=[
            pl.BlockSpec(memory_space=pltpu.SMEM),
            pl.BlockSpec((D_X, BB, N), lambda b: (0, b, 0)),
            pl.BlockSpec((D_OUT, D_X + D_T), lambda b: (0, 0)),
            pl.BlockSpec((1, D_OUT), lambda b: (0, 0)),
        ],
        out_specs=pl.BlockSpec((BB, D_OUT, N), lambda b: (b, 0, 0)),
        out_shape=jax.ShapeDtypeStruct((B, D_OUT, N), jnp.float32),
        compiler_params=pltpu.CompilerParams(
            dimension_semantics=("arbitrary",),
        ),
    )(t, xT, W1, b1[None, :])
    return jnp.transpose(outp, (0, 2, 1))


# speedup vs baseline: 1.5826x; 1.5826x over previous
import math

import jax
import jax.numpy as jnp
from jax.experimental import pallas as pl
from jax.experimental.pallas import tpu as pltpu

B = 32
N = 256
D_X = 50
D_T = 14
D_OUT = 64
HALF = D_T // 2
_FREQ_SCALE = -(math.log(10000.0) / (HALF - 1))

BB = 16


def _fwd_kernel(t_ref, x_ref, w1_ref, b1_ref, o_ref):
    step = pl.program_id(0)
    w1 = w1_ref[...]
    wx = w1[:, :D_X]
    wt = w1[:, D_X:]
    b1_col = jnp.transpose(b1_ref[...], (1, 0))
    i = jax.lax.broadcasted_iota(jnp.int32, (HALF, 1), 0).astype(jnp.float32)
    freqs = jnp.exp(i * _FREQ_SCALE)
    t_row = jnp.concatenate(
        [jnp.full((1, 1), t_ref[step * BB + j], jnp.float32) for j in range(BB)],
        axis=1,
    )
    args = freqs * t_row
    te_mat = jnp.concatenate([jnp.sin(args), jnp.cos(args)], axis=0)
    cc_mat = (
        jax.lax.dot_general(
            wt, te_mat, (((1,), (0,)), ((), ())),
            preferred_element_type=jnp.float32,
        )
        + b1_col
    )
    for j in range(BB):
        y = jax.lax.dot_general(
            wx, x_ref[:, j, :], (((1,), (0,)), ((), ())),
            preferred_element_type=jnp.float32,
        )
        o_ref[j, :, :] = y + jax.lax.slice(cc_mat, (0, j), (D_OUT, j + 1))


def kernel(x, t, obj_cond, edge_cond_in, relation_cond_in, W1, b1):
    xT = jnp.transpose(x, (2, 0, 1))
    outp = pl.pallas_call(
        _fwd_kernel,
        grid=(B // BB,),
        in_specs=[
            pl.BlockSpec(memory_space=pltpu.SMEM),
            pl.BlockSpec((D_X, BB, N), lambda b: (0, b, 0)),
            pl.BlockSpec((D_OUT, D_X + D_T), lambda b: (0, 0)),
            pl.BlockSpec((1, D_OUT), lambda b: (0, 0)),
        ],
        out_specs=pl.BlockSpec((BB, D_OUT, N), lambda b: (b, 0, 0)),
        out_shape=jax.ShapeDtypeStruct((B, D_OUT, N), jnp.float32),
        compiler_params=pltpu.CompilerParams(
            dimension_semantics=("arbitrary",),
        ),
    )(t, xT, W1, b1[None, :])
    return jnp.transpose(outp, (0, 2, 1))


# speedup vs baseline: 1.6239x; 1.0261x over previous
import math

import jax
import jax.numpy as jnp
from jax.experimental import pallas as pl
from jax.experimental.pallas import tpu as pltpu

B = 32
N = 256
D_X = 50
D_T = 14
D_OUT = 64
HALF = D_T // 2
_FREQ_SCALE = -(math.log(10000.0) / (HALF - 1))

BB = 16


def _fwd_kernel(t_ref, x_ref, w1_ref, b1_ref, o_ref):
    step = pl.program_id(0)
    w1 = w1_ref[...]
    wx = w1[:, :D_X]
    wt = w1[:, D_X:]
    b1_col = jnp.transpose(b1_ref[...], (1, 0))
    i = jax.lax.broadcasted_iota(jnp.int32, (HALF, 1), 0).astype(jnp.float32)
    freqs = jnp.exp(i * _FREQ_SCALE)
    t_row = jnp.concatenate(
        [jnp.full((1, 1), t_ref[step * BB + j], jnp.float32) for j in range(BB)],
        axis=1,
    )
    args = freqs * t_row
    te_mat = jnp.concatenate([jnp.sin(args), jnp.cos(args)], axis=0)
    cc_mat = (
        jax.lax.dot_general(
            wt, te_mat, (((1,), (0,)), ((), ())),
            preferred_element_type=jnp.float32,
        )
        + b1_col
    )
    y = jax.lax.dot_general(
        wx, x_ref[...].reshape(D_X, BB * N), (((1,), (0,)), ((), ())),
        preferred_element_type=jnp.float32,
    )
    for j in range(BB):
        o_ref[j, :, :] = jax.lax.slice(
            y, (0, j * N), (D_OUT, (j + 1) * N)
        ) + jax.lax.slice(cc_mat, (0, j), (D_OUT, j + 1))


def kernel(x, t, obj_cond, edge_cond_in, relation_cond_in, W1, b1):
    xT = jnp.transpose(x, (2, 0, 1))
    outp = pl.pallas_call(
        _fwd_kernel,
        grid=(B // BB,),
        in_specs=[
            pl.BlockSpec(memory_space=pltpu.SMEM),
            pl.BlockSpec((D_X, BB, N), lambda b: (0, b, 0)),
            pl.BlockSpec((D_OUT, D_X + D_T), lambda b: (0, 0)),
            pl.BlockSpec((1, D_OUT), lambda b: (0, 0)),
        ],
        out_specs=pl.BlockSpec((BB, D_OUT, N), lambda b: (b, 0, 0)),
        out_shape=jax.ShapeDtypeStruct((B, D_OUT, N), jnp.float32),
        compiler_params=pltpu.CompilerParams(
            dimension_semantics=("arbitrary",),
        ),
    )(t, xT, W1, b1[None, :])
    return jnp.transpose(outp, (0, 2, 1))
